# Initial kernel scaffold; baseline (speedup 1.0000x reference)
import jax
import jax.numpy as jnp
from jax.experimental import pallas as pl


def kernel(pos_feat, pos_labels, ious_pos, neg_feat0, neg_feat1, neg_group_ids, neg_obj_mask):
    raise NotImplementedError("write your pallas kernel here")


# speedup vs baseline: 5.0940x; 5.0940x over previous
import jax
import jax.numpy as jnp
from jax.experimental import pallas as pl
from jax.experimental.pallas import tpu as pltpu

_TEMP = 0.2
_IOU_TH = 0.5
_P = 2048
_N0 = 2048
_N1 = 2048
_K = 128
_M = _P + _N0 + _N1
_R = _P + _N0
_BR = 256
_CC = 512
_NB = _R // _BR


def _loss_kernel(rows_ref, featT_ref, clsr_ref, clsc_ref, ious_ref, out_ref):
    rb = pl.program_id(0)
    rows = rows_ref[...]
    rows_bf = (rows * (1.0 / _TEMP)).astype(jnp.bfloat16)
    clsr = clsr_ref[...]
    g_row = rb * _BR + jax.lax.broadcasted_iota(jnp.int32, (_BR, 1), 0)

    m = None
    s = None
    acc = jnp.zeros((_BR, 1), jnp.float32)
    cnt = jnp.zeros((_BR, 1), jnp.float32)
    for c in range(_M // _CC):
        chunk = featT_ref[:, c * _CC:(c + 1) * _CC]
        sim = jax.lax.dot_general(
            rows_bf, chunk, (((1,), (0,)), ((), ())),
            preferred_element_type=jnp.float32)
        cm = jnp.max(sim, axis=1, keepdims=True)
        col_ids = c * _CC + jax.lax.broadcasted_iota(jnp.int32, (1, _CC), 1)
        offd = col_ids != g_row
        if m is None:
            new_m = cm
        else:
            new_m = jnp.maximum(m, cm)
        es = jnp.where(offd, jnp.exp(sim - new_m), 0.0)
        chunk_s = jnp.sum(es, axis=1, keepdims=True)
        if s is None:
            s = chunk_s
        else:
            s = s * jnp.exp(m - new_m) + chunk_s
        m = new_m
        eqm = (clsr == clsc_ref[:, c * _CC:(c + 1) * _CC]) & offd
        acc = acc + jnp.sum(jnp.where(eqm, sim, 0.0), axis=1, keepdims=True)
        cnt = cnt + jnp.sum(jnp.where(eqm, 1.0, 0.0), axis=1, keepdims=True)

    log_denom = m + jnp.log(s)
    plp = (acc - cnt * log_denom) / (cnt + 1.0)
    keep = jnp.where(ious_ref[...] >= _IOU_TH, 1.0, 0.0)
    lsum = jnp.sum(keep * plp)
    ksum = jnp.sum(keep)
    lane = jax.lax.broadcasted_iota(jnp.int32, (1, 1, 128), 2)
    out_ref[...] = jnp.where(lane == 0, lsum, jnp.where(lane == 1, ksum, 0.0))


def _reduce_kernel(parts_ref, out_ref):
    tot = jnp.sum(parts_ref[...], axis=0)
    l = tot[:, 0:1]
    k = tot[:, 1:2]
    out_ref[...] = -(l / k)


def kernel(pos_feat, pos_labels, ious_pos, neg_feat0, neg_feat1,
           neg_group_ids, neg_obj_mask):
    labels = pos_labels.astype(jnp.int32)
    gids = neg_group_ids.astype(jnp.int32)
    omask = neg_obj_mask.astype(jnp.int32)

    ones_before = jnp.cumsum(omask) - omask
    n_ones = jnp.sum(omask)
    idx = jnp.arange(_N1, dtype=jnp.int32)
    pos_of = jnp.where(omask == 1, ones_before, n_ones + (idx - ones_before))
    g_range = jnp.arange(1, 16, dtype=jnp.int32)
    starts = jnp.sum((gids[None, :] < g_range[:, None]).astype(jnp.int32),
                     axis=1)
    pg = jnp.sum((pos_of[None, :] >= starts[:, None]).astype(jnp.int32),
                 axis=0)

    cls = jnp.concatenate([labels, gids + 80, pg + 80])
    feat = jnp.concatenate([pos_feat, neg_feat0, neg_feat1], axis=0)
    featT = feat.T.astype(jnp.bfloat16)
    rows = feat[:_R]
    clsr = cls[:_R].reshape(_R, 1)
    clsc = cls.reshape(1, _M)
    ious_full = jnp.concatenate(
        [ious_pos, jnp.ones((_N0,), jnp.float32)]).reshape(_R, 1)

    parts = pl.pallas_call(
        _loss_kernel,
        grid=(_NB,),
        in_specs=[
            pl.BlockSpec((_BR, _K), lambda i: (i, 0)),
            pl.BlockSpec((_K, _M), lambda i: (0, 0)),
            pl.BlockSpec((_BR, 1), lambda i: (i, 0)),
            pl.BlockSpec((1, _M), lambda i: (0, 0)),
            pl.BlockSpec((_BR, 1), lambda i: (i, 0)),
        ],
        out_specs=pl.BlockSpec((1, 1, 128), lambda i: (i, 0, 0)),
        out_shape=jax.ShapeDtypeStruct((_NB, 1, 128), jnp.float32),
        compiler_params=pltpu.CompilerParams(
            dimension_semantics=("parallel",),
            vmem_limit_bytes=48 * 1024 * 1024,
        ),
    )(rows, featT, clsr, clsc, ious_full)

    out = pl.pallas_call(
        _reduce_kernel,
        out_shape=jax.ShapeDtypeStruct((1, 1), jnp.float32),
    )(parts)
    return jnp.reshape(out, ())


# speedup vs baseline: 8.0399x; 1.5783x over previous
import jax
import jax.numpy as jnp
from jax.experimental import pallas as pl
from jax.experimental.pallas import tpu as pltpu

_TEMP = 0.2
_IOU_TH = 0.5
_LOG2E = 1.4426950408889634
_LN2 = 0.6931471805599453
_P = 2048
_N0 = 2048
_N1 = 2048
_K = 128
_M = _P + _N0 + _N1
_R = _P + _N0
_BR = 256
_CC = 512
_NB = _R // _BR
_SEG = _P // _CC


def _stats_kernel(cls_ref, p_ref, n0_ref, n1_ref, s_ref, n_ref):
    lane = jax.lax.broadcasted_iota(jnp.int32, (1, 128), 1)
    cls_col = cls_ref[...]
    eq = cls_col == lane
    oh_f = jnp.where(eq, 1.0, 0.0)
    oh_bf = oh_f.astype(jnp.bfloat16)
    s = jnp.zeros((128, 128), jnp.float32)
    for seg, f in ((0, p_ref), (1, n0_ref), (2, n1_ref)):
        s = s + jax.lax.dot_general(
            oh_bf[seg * _P:(seg + 1) * _P, :], f[...],
            (((0,), (0,)), ((), ())), preferred_element_type=jnp.float32)
    s_ref[...] = s
    n_ref[...] = jnp.sum(oh_f, axis=0, keepdims=True)


def _loss_kernel(clsr_ref, ious_ref, prow_ref, nrow_ref,
                 pbf_ref, n0bf_ref, n1bf_ref, s_ref, n_ref, out_ref):
    rb = pl.program_id(0)
    is_pos = rb < _NB // 2
    rows = jnp.where(is_pos, prow_ref[...], nrow_ref[...])
    rows_bf = (rows * (_LOG2E / _TEMP)).astype(jnp.bfloat16)
    rows_bf32 = rows_bf.astype(jnp.float32)
    g_row = rb * _BR + jax.lax.broadcasted_iota(jnp.int32, (_BR, 1), 0)

    m = None
    s = None
    srcs = [pbf_ref] * _SEG + [n0bf_ref] * _SEG + [n1bf_ref] * _SEG
    for c in range(_M // _CC):
        lo = (c % _SEG) * _CC
        chunk = srcs[c][lo:lo + _CC, :]
        sim = jax.lax.dot_general(
            rows_bf, chunk, (((1,), (1,)), ((), ())),
            preferred_element_type=jnp.float32)
        cm = jnp.max(sim, axis=1, keepdims=True)
        col_ids = c * _CC + jax.lax.broadcasted_iota(jnp.int32, (1, _CC), 1)
        offd = col_ids != g_row
        new_m = cm if m is None else jnp.maximum(m, cm)
        es = jnp.where(offd, jnp.exp2(sim - new_m), 0.0)
        chunk_s = jnp.sum(es, axis=1, keepdims=True)
        s = chunk_s if s is None else s * jnp.exp2(m - new_m) + chunk_s
        m = new_m

    lane = jax.lax.broadcasted_iota(jnp.int32, (1, 128), 1)
    oh_rows = jnp.where(clsr_ref[...] == lane, 1.0, 0.0)
    z = jax.lax.dot_general(oh_rows, s_ref[...], (((1,), (0,)), ((), ())),
                            preferred_element_type=jnp.float32)
    a_full = jnp.sum(rows_bf32 * z, axis=1, keepdims=True)
    diag = jnp.sum(rows_bf32 * rows.astype(jnp.bfloat16).astype(jnp.float32),
                   axis=1, keepdims=True)
    cnt = jnp.sum(oh_rows * n_ref[...], axis=1, keepdims=True)

    log_denom = m + jnp.log2(s)
    plp = (a_full - diag - (cnt - 1.0) * log_denom) / cnt
    keep = jnp.where(ious_ref[...] >= _IOU_TH, 1.0, 0.0)
    lsum = jnp.sum(keep * plp)
    ksum = jnp.sum(keep)
    olane = jax.lax.broadcasted_iota(jnp.int32, (1, 1, 128), 2)
    out_ref[...] = jnp.where(olane == 0, lsum, jnp.where(olane == 1, ksum, 0.0))


def _reduce_kernel(parts_ref, out_ref):
    tot = jnp.sum(parts_ref[...], axis=0)
    l = tot[:, 0:1]
    k = tot[:, 1:2]
    out_ref[...] = -(_LN2 * l) / k


def kernel(pos_feat, pos_labels, ious_pos, neg_feat0, neg_feat1,
           neg_group_ids, neg_obj_mask):
    labels = pos_labels.astype(jnp.int32)
    gids = neg_group_ids.astype(jnp.int32)
    omask = neg_obj_mask.astype(jnp.int32)

    ones_before = jnp.cumsum(omask) - omask
    n_ones = jnp.sum(omask)
    idx = jnp.arange(_N1, dtype=jnp.int32)
    pos_of = jnp.where(omask == 1, ones_before, n_ones + (idx - ones_before))
    g_range = jnp.arange(1, 16, dtype=jnp.int32)
    starts = jnp.sum((gids[None, :] < g_range[:, None]).astype(jnp.int32),
                     axis=1)
    pg = jnp.sum((pos_of[None, :] >= starts[:, None]).astype(jnp.int32),
                 axis=0)

    cls = jnp.concatenate([labels, gids + 80, pg + 80])
    cls_col = cls.reshape(_M, 1)
    clsr = cls[:_R].reshape(_R, 1)
    ious_full = jnp.concatenate(
        [ious_pos, jnp.ones((_N0,), jnp.float32)]).reshape(_R, 1)
    p_bf = pos_feat.astype(jnp.bfloat16)
    n0_bf = neg_feat0.astype(jnp.bfloat16)
    n1_bf = neg_feat1.astype(jnp.bfloat16)

    s_cls, n_cls = pl.pallas_call(
        _stats_kernel,
        out_shape=(jax.ShapeDtypeStruct((128, 128), jnp.float32),
                   jax.ShapeDtypeStruct((1, 128), jnp.float32)),
    )(cls_col, p_bf, n0_bf, n1_bf)

    nh = _NB // 2
    parts = pl.pallas_call(
        _loss_kernel,
        grid=(_NB,),
        in_specs=[
            pl.BlockSpec((_BR, 1), lambda i: (i, 0)),
            pl.BlockSpec((_BR, 1), lambda i: (i, 0)),
            pl.BlockSpec((_BR, _K), lambda i: (jnp.minimum(i, nh - 1), 0)),
            pl.BlockSpec((_BR, _K), lambda i: (jnp.maximum(i, nh) - nh, 0)),
            pl.BlockSpec((_P, _K), lambda i: (0, 0)),
            pl.BlockSpec((_N0, _K), lambda i: (0, 0)),
            pl.BlockSpec((_N1, _K), lambda i: (0, 0)),
            pl.BlockSpec((128, 128), lambda i: (0, 0)),
            pl.BlockSpec((1, 128), lambda i: (0, 0)),
        ],
        out_specs=pl.BlockSpec((1, 1, 128), lambda i: (i, 0, 0)),
        out_shape=jax.ShapeDtypeStruct((_NB, 1, 128), jnp.float32),
        compiler_params=pltpu.CompilerParams(
            dimension_semantics=("parallel",),
            vmem_limit_bytes=48 * 1024 * 1024,
        ),
    )(clsr, ious_full, pos_feat, neg_feat0, p_bf, n0_bf, n1_bf, s_cls, n_cls)

    out = pl.pallas_call(
        _reduce_kernel,
        out_shape=jax.ShapeDtypeStruct((1, 1), jnp.float32),
    )(parts)
    return jnp.reshape(out, ())


# speedup vs baseline: 9.2019x; 1.1445x over previous
import jax
import jax.numpy as jnp
from jax.experimental import pallas as pl
from jax.experimental.pallas import tpu as pltpu

_TEMP = 0.2
_IOU_TH = 0.5
_LOG2E = 1.4426950408889634
_LN2 = 0.6931471805599453
_P = 2048
_N0 = 2048
_N1 = 2048
_K = 128
_M = _P + _N0 + _N1
_R = _P + _N0
_BR = 256
_CC = 512
_NB = _R // _BR
_SEG = _P // _CC


def _stats_kernel(cls_ref, p_ref, n0_ref, n1_ref, s_ref, n_ref):
    lane = jax.lax.broadcasted_iota(jnp.int32, (1, 128), 1)
    cls_col = cls_ref[...]
    eq = cls_col == lane
    oh_f = jnp.where(eq, 1.0, 0.0)
    oh_bf = oh_f.astype(jnp.bfloat16)
    s = jnp.zeros((128, 128), jnp.float32)
    for seg, f in ((0, p_ref), (1, n0_ref), (2, n1_ref)):
        s = s + jax.lax.dot_general(
            oh_bf[seg * _P:(seg + 1) * _P, :], f[...],
            (((0,), (0,)), ((), ())), preferred_element_type=jnp.float32)
    s_ref[...] = s
    n_ref[...] = jnp.sum(oh_f, axis=0, keepdims=True)


def _loss_kernel(clsr_ref, ious_ref, prow_ref, nrow_ref,
                 pbf_ref, n0bf_ref, n1bf_ref, s_ref, n_ref, out_ref):
    rb = pl.program_id(0)
    is_pos = rb < _NB // 2
    rows = jnp.where(is_pos, prow_ref[...], nrow_ref[...])
    rows_bf = (rows * (_LOG2E / _TEMP)).astype(jnp.bfloat16)
    rows_bf32 = rows_bf.astype(jnp.float32)

    m = None
    s = None
    srcs = [pbf_ref] * _SEG + [n0bf_ref] * _SEG + [n1bf_ref] * _SEG
    for c in range(_M // _CC):
        lo = (c % _SEG) * _CC
        chunk = srcs[c][lo:lo + _CC, :]
        sim = jax.lax.dot_general(
            rows_bf, chunk, (((1,), (1,)), ((), ())),
            preferred_element_type=jnp.float32)
        cm = jnp.max(sim, axis=1, keepdims=True)
        new_m = cm if m is None else jnp.maximum(m, cm)
        chunk_s = jnp.sum(jnp.exp2(sim - new_m), axis=1, keepdims=True)
        s = chunk_s if s is None else s * jnp.exp2(m - new_m) + chunk_s
        m = new_m

    lane = jax.lax.broadcasted_iota(jnp.int32, (1, 128), 1)
    oh_rows = jnp.where(clsr_ref[...] == lane, 1.0, 0.0)
    z = jax.lax.dot_general(oh_rows, s_ref[...], (((1,), (0,)), ((), ())),
                            preferred_element_type=jnp.float32)
    a_full = jnp.sum(rows_bf32 * z, axis=1, keepdims=True)
    diag = jnp.sum(rows_bf32 * rows.astype(jnp.bfloat16).astype(jnp.float32),
                   axis=1, keepdims=True)
    cnt = jnp.sum(oh_rows * n_ref[...], axis=1, keepdims=True)

    s = s - jnp.exp2(diag - m)
    log_denom = m + jnp.log2(s)
    plp = (a_full - diag - (cnt - 1.0) * log_denom) / cnt
    keep = jnp.where(ious_ref[...] >= _IOU_TH, 1.0, 0.0)
    lsum = jnp.sum(keep * plp)
    ksum = jnp.sum(keep)
    olane = jax.lax.broadcasted_iota(jnp.int32, (1, 1, 128), 2)
    out_ref[...] = jnp.where(olane == 0, lsum, jnp.where(olane == 1, ksum, 0.0))


def _reduce_kernel(parts_ref, out_ref):
    tot = jnp.sum(parts_ref[...], axis=0)
    l = tot[:, 0:1]
    k = tot[:, 1:2]
    out_ref[...] = -(_LN2 * l) / k


def kernel(pos_feat, pos_labels, ious_pos, neg_feat0, neg_feat1,
           neg_group_ids, neg_obj_mask):
    labels = pos_labels.astype(jnp.int32)
    gids = neg_group_ids.astype(jnp.int32)
    omask = neg_obj_mask.astype(jnp.int32)

    ones_before = jnp.cumsum(omask) - omask
    n_ones = jnp.sum(omask)
    idx = jnp.arange(_N1, dtype=jnp.int32)
    pos_of = jnp.where(omask == 1, ones_before, n_ones + (idx - ones_before))
    g_range = jnp.arange(1, 16, dtype=jnp.int32)
    starts = jnp.sum((gids[None, :] < g_range[:, None]).astype(jnp.int32),
                     axis=1)
    pg = jnp.sum((pos_of[None, :] >= starts[:, None]).astype(jnp.int32),
                 axis=0)

    cls = jnp.concatenate([labels, gids + 80, pg + 80])
    cls_col = cls.reshape(_M, 1)
    clsr = cls[:_R].reshape(_R, 1)
    ious_full = jnp.concatenate(
        [ious_pos, jnp.ones((_N0,), jnp.float32)]).reshape(_R, 1)
    p_bf = pos_feat.astype(jnp.bfloat16)
    n0_bf = neg_feat0.astype(jnp.bfloat16)
    n1_bf = neg_feat1.astype(jnp.bfloat16)

    s_cls, n_cls = pl.pallas_call(
        _stats_kernel,
        out_shape=(jax.ShapeDtypeStruct((128, 128), jnp.float32),
                   jax.ShapeDtypeStruct((1, 128), jnp.float32)),
    )(cls_col, p_bf, n0_bf, n1_bf)

    nh = _NB // 2
    parts = pl.pallas_call(
        _loss_kernel,
        grid=(_NB,),
        in_specs=[
            pl.BlockSpec((_BR, 1), lambda i: (i, 0)),
            pl.BlockSpec((_BR, 1), lambda i: (i, 0)),
            pl.BlockSpec((_BR, _K), lambda i: (jnp.minimum(i, nh - 1), 0)),
            pl.BlockSpec((_BR, _K), lambda i: (jnp.maximum(i, nh) - nh, 0)),
            pl.BlockSpec((_P, _K), lambda i: (0, 0)),
            pl.BlockSpec((_N0, _K), lambda i: (0, 0)),
            pl.BlockSpec((_N1, _K), lambda i: (0, 0)),
            pl.BlockSpec((128, 128), lambda i: (0, 0)),
            pl.BlockSpec((1, 128), lambda i: (0, 0)),
        ],
        out_specs=pl.BlockSpec((1, 1, 128), lambda i: (i, 0, 0)),
        out_shape=jax.ShapeDtypeStruct((_NB, 1, 128), jnp.float32),
        compiler_params=pltpu.CompilerParams(
            dimension_semantics=("parallel",),
            vmem_limit_bytes=48 * 1024 * 1024,
        ),
    )(clsr, ious_full, pos_feat, neg_feat0, p_bf, n0_bf, n1_bf, s_cls, n_cls)

    out = pl.pallas_call(
        _reduce_kernel,
        out_shape=jax.ShapeDtypeStruct((1, 1), jnp.float32),
    )(parts)
    return jnp.reshape(out, ())
